# Initial kernel scaffold; baseline (speedup 1.0000x reference)
import jax
import jax.numpy as jnp
from jax import lax
from jax.experimental import pallas as pl
from jax.experimental.pallas import tpu as pltpu
from jax.experimental.pallas import tpu_sc as plsc

N = 10000
NPAD = 10240
E = 160000
NG = 64
NC = 2
NS = 16
NW = NC * NS
CH = 128
NCHUNK = 40
EPT = CH * NCHUNK
EPAD = EPT * NW
RPT = NPAD // NS
ZOFF = NPAD - CH


def _deg_body(dst_hbm, aux_hbm, out_hbm, dst_v, ones_v, deg_sh):
    c = lax.axis_index("c")
    s = lax.axis_index("s")
    wid = c * NS + s
    pltpu.sync_copy(dst_hbm.at[wid], dst_v)
    pltpu.sync_copy(aux_hbm.at[pl.ds(0, CH)], ones_v)
    pltpu.sync_copy(aux_hbm.at[pl.ds(CH, RPT)], deg_sh.at[pl.ds(s * RPT, RPT)])
    plsc.subcore_barrier()

    @pl.loop(0, NCHUNK)
    def _chunk(j):
        pltpu.sync_copy(ones_v, deg_sh.at[dst_v.at[j]], add=True)

    plsc.subcore_barrier()
    pltpu.sync_copy(deg_sh.at[pl.ds(s * RPT, RPT)],
                    out_hbm.at[c, pl.ds(s * RPT, RPT)])


_deg_call = pl.kernel(
    _deg_body,
    out_type=jax.ShapeDtypeStruct((NC, NPAD), jnp.float32),
    mesh=plsc.VectorSubcoreMesh(core_axis_name="c", subcore_axis_name="s"),
    scratch_types=[
        pltpu.VMEM((NCHUNK, CH), jnp.int32),
        pltpu.VMEM((CH,), jnp.float32),
        pltpu.VMEM_SHARED((NPAD,), jnp.float32),
    ],
)


def _agg_body(xs_hbm, src_hbm, dst_hbm, out_hbm, src_v, dst_v, rows_v, agg_sh):
    c = lax.axis_index("c")
    s = lax.axis_index("s")
    wid = c * NS + s
    pltpu.sync_copy(src_hbm.at[wid], src_v)
    pltpu.sync_copy(dst_hbm.at[wid], dst_v)
    for k in range(RPT // CH):
        pltpu.sync_copy(xs_hbm.at[pl.ds(ZOFF, CH)],
                        agg_sh.at[pl.ds(s * RPT + k * CH, CH)])
    plsc.subcore_barrier()

    @pl.loop(0, NCHUNK)
    def _chunk(j):
        pltpu.sync_copy(xs_hbm.at[src_v.at[j]], rows_v)
        pltpu.sync_copy(rows_v, agg_sh.at[dst_v.at[j]], add=True)

    plsc.subcore_barrier()
    pltpu.sync_copy(agg_sh.at[pl.ds(s * RPT, RPT)],
                    out_hbm.at[c, pl.ds(s * RPT, RPT)])


def _make_agg(F):
    return pl.kernel(
        _agg_body,
        out_type=jax.ShapeDtypeStruct((NC, NPAD, F), jnp.float32),
        mesh=plsc.VectorSubcoreMesh(core_axis_name="c", subcore_axis_name="s"),
        scratch_types=[
            pltpu.VMEM((NCHUNK, CH), jnp.int32),
            pltpu.VMEM((NCHUNK, CH), jnp.int32),
            pltpu.VMEM((CH, F), jnp.float32),
            pltpu.VMEM_SHARED((NPAD, F), jnp.float32),
        ],
    )


_agg32 = _make_agg(32)
_agg64 = _make_agg(64)


def _tc1_body(x_ref, w1_ref, p_ref, xw1_ref, dinv_ref, xs1_ref):
    xw1 = jnp.dot(x_ref[...], w1_ref[...], preferred_element_type=jnp.float32)
    p = p_ref[...]
    deg = p[0] + p[1] + 1.0
    dinv = lax.rsqrt(deg)[:, None]
    row = lax.broadcasted_iota(jnp.int32, (NPAD, 1), 0)
    dinv = jnp.where(row < N, dinv, 0.0)
    xw1_ref[...] = xw1
    dinv_ref[...] = dinv
    xs1_ref[...] = xw1 * dinv


_tc1 = pl.pallas_call(
    _tc1_body,
    out_shape=(
        jax.ShapeDtypeStruct((NPAD, 32), jnp.float32),
        jax.ShapeDtypeStruct((NPAD, 1), jnp.float32),
        jax.ShapeDtypeStruct((NPAD, 32), jnp.float32),
    ),
)


def _tc2_body(q_ref, xw1_ref, dinv_ref, b1_ref, w2_ref, xw2_ref, xs2_ref):
    q = q_ref[...]
    dinv = dinv_ref[...]
    pre = dinv * (q[0] + q[1]) + dinv * dinv * xw1_ref[...] + b1_ref[...]
    h1 = jnp.where(pre >= 0, pre, 0.1 * pre)
    xw2 = jnp.dot(h1, w2_ref[...], preferred_element_type=jnp.float32)
    xw2_ref[...] = xw2
    xs2_ref[...] = xw2 * dinv


_tc2 = pl.pallas_call(
    _tc2_body,
    out_shape=(
        jax.ShapeDtypeStruct((NPAD, 64), jnp.float32),
        jax.ShapeDtypeStruct((NPAD, 64), jnp.float32),
    ),
)


def _tc3_body(r_ref, xw2_ref, dinv_ref, b2_ref, batch_ref,
              l1w_ref, l1b_ref, l2w_ref, l2b_ref, y_ref):
    rr = r_ref[...]
    dinv = dinv_ref[...]
    h2 = dinv * (rr[0] + rr[1]) + dinv * dinv * xw2_ref[...] + b2_ref[...]
    b = batch_ref[...]

    def _seg(g, pooled):
        v = jnp.where(b == g, h2, -jnp.inf)
        return lax.dynamic_update_slice(
            pooled, jnp.max(v, axis=0, keepdims=True), (g, 0))

    pooled = lax.fori_loop(0, NG, _seg, jnp.full((NG, 64), -jnp.inf, jnp.float32))
    t = jnp.dot(pooled, l1w_ref[...], preferred_element_type=jnp.float32) + l1b_ref[...]
    t = jnp.where(t >= 0, t, 0.1 * t)
    y_ref[...] = jnp.dot(t, l2w_ref[...], preferred_element_type=jnp.float32) + l2b_ref[...]


_tc3 = pl.pallas_call(
    _tc3_body,
    out_shape=jax.ShapeDtypeStruct((NG, 32), jnp.float32),
)


def kernel(x, edge_index, batch, W1, b1, W2, b2, L1W, L1b, L2W, L2b):
    edge_index = edge_index.astype(jnp.int32)
    src = edge_index[0]
    dst = edge_index[1]
    pad = jnp.full((EPAD - E,), N, dtype=jnp.int32)
    src_r = jnp.concatenate([src, pad]).reshape(NW, NCHUNK, CH)
    dst_r = jnp.concatenate([dst, pad]).reshape(NW, NCHUNK, CH)
    aux = jnp.concatenate([jnp.ones((CH,), jnp.float32),
                           jnp.zeros((RPT,), jnp.float32)])
    x_pad = jnp.pad(x, ((0, NPAD - N), (0, 0)))
    batch_pad = jnp.concatenate(
        [batch.astype(jnp.int32), jnp.full((NPAD - N,), NG, jnp.int32)])[:, None]

    p = _deg_call(dst_r, aux)
    xw1, dinv, xs1 = _tc1(x_pad, W1, p)
    q = _agg32(xs1, src_r, dst_r)
    xw2, xs2 = _tc2(q, xw1, dinv, b1, W2)
    r = _agg64(xs2, src_r, dst_r)
    y = _tc3(r, xw2, dinv, b2, batch_pad, L1W, L1b, L2W, L2b)
    return y


# speedup vs baseline: 11.2103x; 11.2103x over previous
import jax
import jax.numpy as jnp
from jax import lax
from jax.experimental import pallas as pl
from jax.experimental.pallas import tpu as pltpu
from jax.experimental.pallas import tpu_sc as plsc

N = 10000
NPAD = 10240
E = 160000
NG = 64
NC = 2
NS = 16
NW = NC * NS
CH = 128
NCHUNK = 40
EPT = CH * NCHUNK
EPAD = EPT * NW
RPT = NPAD // NS
ZOFF = NPAD - CH


def _deg_body(dst_hbm, aux_hbm, out_hbm, dst_v, ones_v, deg_sh):
    c = lax.axis_index("c")
    s = lax.axis_index("s")
    wid = c * NS + s
    pltpu.sync_copy(dst_hbm.at[wid], dst_v)
    pltpu.sync_copy(aux_hbm.at[pl.ds(0, CH)], ones_v)
    pltpu.sync_copy(aux_hbm.at[pl.ds(CH, RPT)], deg_sh.at[pl.ds(s * RPT, RPT)])
    plsc.subcore_barrier()

    @pl.loop(0, NCHUNK)
    def _chunk(j):
        pltpu.sync_copy(ones_v, deg_sh.at[dst_v.at[j]], add=True)

    plsc.subcore_barrier()
    pltpu.sync_copy(deg_sh.at[pl.ds(s * RPT, RPT)],
                    out_hbm.at[c, pl.ds(s * RPT, RPT)])


_deg_call = pl.kernel(
    _deg_body,
    out_type=jax.ShapeDtypeStruct((NC, NPAD), jnp.float32),
    mesh=plsc.VectorSubcoreMesh(core_axis_name="c", subcore_axis_name="s"),
    scratch_types=[
        pltpu.VMEM((NCHUNK, CH), jnp.int32),
        pltpu.VMEM((CH,), jnp.float32),
        pltpu.VMEM_SHARED((NPAD,), jnp.float32),
    ],
)


def _agg_body(xs_hbm, src_hbm, dst_hbm, out_hbm, src_v, dst_v, rows_v, agg_sh):
    c = lax.axis_index("c")
    s = lax.axis_index("s")
    wid = c * NS + s
    pltpu.sync_copy(src_hbm.at[wid], src_v)
    pltpu.sync_copy(dst_hbm.at[wid], dst_v)
    for k in range(RPT // CH):
        pltpu.sync_copy(xs_hbm.at[pl.ds(ZOFF, CH)],
                        agg_sh.at[pl.ds(s * RPT + k * CH, CH)])
    plsc.subcore_barrier()

    @pl.loop(0, NCHUNK)
    def _chunk(j):
        pltpu.sync_copy(xs_hbm.at[src_v.at[j]], rows_v)
        pltpu.sync_copy(rows_v, agg_sh.at[dst_v.at[j]], add=True)

    plsc.subcore_barrier()
    pltpu.sync_copy(agg_sh.at[pl.ds(s * RPT, RPT)],
                    out_hbm.at[c, pl.ds(s * RPT, RPT)])


def _make_agg(F):
    return pl.kernel(
        _agg_body,
        out_type=jax.ShapeDtypeStruct((NC, NPAD, F), jnp.float32),
        mesh=plsc.VectorSubcoreMesh(core_axis_name="c", subcore_axis_name="s"),
        compiler_params=pltpu.CompilerParams(use_tc_tiling_on_sc=False),
        scratch_types=[
            pltpu.VMEM((NCHUNK, CH), jnp.int32),
            pltpu.VMEM((NCHUNK, CH), jnp.int32),
            pltpu.VMEM((CH, F), jnp.float32),
            pltpu.VMEM_SHARED((NPAD, F), jnp.float32),
        ],
    )


_agg32 = _make_agg(32)
_agg64 = _make_agg(64)


def _tc1_body(x_ref, w1_ref, p_ref, xw1_ref, dinv_ref, xs1_ref):
    xw1 = jnp.dot(x_ref[...], w1_ref[...], preferred_element_type=jnp.float32)
    p = p_ref[...]
    deg = p[0] + p[1] + 1.0
    dinv = lax.rsqrt(deg)[:, None]
    row = lax.broadcasted_iota(jnp.int32, (NPAD, 1), 0)
    dinv = jnp.where(row < N, dinv, 0.0)
    xw1_ref[...] = xw1
    dinv_ref[...] = dinv
    xs1_ref[...] = xw1 * dinv


_tc1 = pl.pallas_call(
    _tc1_body,
    out_shape=(
        jax.ShapeDtypeStruct((NPAD, 32), jnp.float32),
        jax.ShapeDtypeStruct((NPAD, 1), jnp.float32),
        jax.ShapeDtypeStruct((NPAD, 32), jnp.float32),
    ),
)


def _tc2_body(q_ref, xw1_ref, dinv_ref, b1_ref, w2_ref, xw2_ref, xs2_ref):
    q = q_ref[...]
    dinv = dinv_ref[...]
    pre = dinv * (q[0] + q[1]) + dinv * dinv * xw1_ref[...] + b1_ref[...]
    h1 = jnp.where(pre >= 0, pre, 0.1 * pre)
    xw2 = jnp.dot(h1, w2_ref[...], preferred_element_type=jnp.float32)
    xw2_ref[...] = xw2
    xs2_ref[...] = xw2 * dinv


_tc2 = pl.pallas_call(
    _tc2_body,
    out_shape=(
        jax.ShapeDtypeStruct((NPAD, 64), jnp.float32),
        jax.ShapeDtypeStruct((NPAD, 64), jnp.float32),
    ),
)


def _tc3_body(r_ref, xw2_ref, dinv_ref, b2_ref, batch_ref,
              l1w_ref, l1b_ref, l2w_ref, l2b_ref, y_ref, pooled_ref):
    rr = r_ref[...]
    dinv = dinv_ref[...]
    h2 = dinv * (rr[0] + rr[1]) + dinv * dinv * xw2_ref[...] + b2_ref[...]
    b = batch_ref[...]

    def _seg(g, carry):
        v = jnp.where(b == g, h2, -jnp.inf)
        pooled_ref[pl.ds(g, 1), :] = jnp.max(v, axis=0, keepdims=True)
        return carry

    lax.fori_loop(0, NG, _seg, 0)
    pooled = pooled_ref[...]
    t = jnp.dot(pooled, l1w_ref[...], preferred_element_type=jnp.float32) + l1b_ref[...]
    t = jnp.where(t >= 0, t, 0.1 * t)
    y_ref[...] = jnp.dot(t, l2w_ref[...], preferred_element_type=jnp.float32) + l2b_ref[...]


_tc3 = pl.pallas_call(
    _tc3_body,
    out_shape=jax.ShapeDtypeStruct((NG, 32), jnp.float32),
    scratch_shapes=[pltpu.VMEM((NG, 64), jnp.float32)],
)


def kernel(x, edge_index, batch, W1, b1, W2, b2, L1W, L1b, L2W, L2b):
    edge_index = edge_index.astype(jnp.int32)
    src = edge_index[0]
    dst = edge_index[1]
    pad = jnp.full((EPAD - E,), N, dtype=jnp.int32)
    src_r = jnp.concatenate([src, pad]).reshape(NW, NCHUNK, CH)
    dst_r = jnp.concatenate([dst, pad]).reshape(NW, NCHUNK, CH)
    aux = jnp.concatenate([jnp.ones((CH,), jnp.float32),
                           jnp.zeros((RPT,), jnp.float32)])
    x_pad = jnp.pad(x, ((0, NPAD - N), (0, 0)))
    batch_pad = jnp.concatenate(
        [batch.astype(jnp.int32), jnp.full((NPAD - N,), NG, jnp.int32)])[:, None]

    p = _deg_call(dst_r, aux)
    xw1, dinv, xs1 = _tc1(x_pad, W1, p)
    q = _agg32(xs1, src_r, dst_r)
    xw2, xs2 = _tc2(q, xw1, dinv, b1, W2)
    r = _agg64(xs2, src_r, dst_r)
    y = _tc3(r, xw2, dinv, b2, batch_pad, L1W, L1b, L2W, L2b)
    return y


# speedup vs baseline: 12.8013x; 1.1419x over previous
import jax
import jax.numpy as jnp
from jax import lax
from jax.experimental import pallas as pl
from jax.experimental.pallas import tpu as pltpu
from jax.experimental.pallas import tpu_sc as plsc

N = 10000
NPAD = 10240
E = 160000
NG = 64
NC = 2
NS = 16
NW = NC * NS
CH = 128
NCHUNK = 40
EPT = CH * NCHUNK
EPAD = EPT * NW
RPT = NPAD // NS
ZOFF = NPAD - CH


def _deg_body(dst_hbm, aux_hbm, out_hbm, dst_v, ones_v, deg_sh):
    c = lax.axis_index("c")
    s = lax.axis_index("s")
    wid = c * NS + s
    pltpu.sync_copy(dst_hbm.at[wid], dst_v)
    pltpu.sync_copy(aux_hbm.at[pl.ds(0, CH)], ones_v)
    pltpu.sync_copy(aux_hbm.at[pl.ds(CH, RPT)], deg_sh.at[pl.ds(s * RPT, RPT)])
    plsc.subcore_barrier()

    @pl.loop(0, NCHUNK)
    def _chunk(j):
        pltpu.sync_copy(ones_v, deg_sh.at[dst_v.at[j]], add=True)

    plsc.subcore_barrier()
    pltpu.sync_copy(deg_sh.at[pl.ds(s * RPT, RPT)],
                    out_hbm.at[c, pl.ds(s * RPT, RPT)])


_deg_call = pl.kernel(
    _deg_body,
    out_type=jax.ShapeDtypeStruct((NC, NPAD), jnp.float32),
    mesh=plsc.VectorSubcoreMesh(core_axis_name="c", subcore_axis_name="s"),
    scratch_types=[
        pltpu.VMEM((NCHUNK, CH), jnp.int32),
        pltpu.VMEM((CH,), jnp.float32),
        pltpu.VMEM_SHARED((NPAD,), jnp.float32),
    ],
)


NBUF = 4


def _agg_body(xs_hbm, src_hbm, dst_hbm, out_hbm, src_v, dst_v, rows_v, agg_sh,
              gsem):
    c = lax.axis_index("c")
    s = lax.axis_index("s")
    wid = c * NS + s
    pltpu.sync_copy(src_hbm.at[wid], src_v)
    pltpu.sync_copy(dst_hbm.at[wid], dst_v)
    for k in range(RPT // CH):
        pltpu.sync_copy(xs_hbm.at[pl.ds(ZOFF, CH)],
                        agg_sh.at[pl.ds(s * RPT + k * CH, CH)])
    plsc.subcore_barrier()

    def _start_gather(j, b):
        pltpu.async_copy(xs_hbm.at[src_v.at[j]], rows_v.at[b], gsem.at[b])

    def _wait_gather(b):
        pltpu.make_async_copy(xs_hbm.at[src_v.at[0]], rows_v.at[b],
                              gsem.at[b]).wait()

    for b in range(NBUF):
        _start_gather(b, b)

    @pl.loop(0, NCHUNK - NBUF, step=NBUF)
    def _round(j0):
        for b in range(NBUF):
            j = j0 + b
            _wait_gather(b)
            pltpu.sync_copy(rows_v.at[b], agg_sh.at[dst_v.at[j]], add=True)
            _start_gather(j + NBUF, b)

    for b in range(NBUF):
        _wait_gather(b)
        pltpu.sync_copy(rows_v.at[b], agg_sh.at[dst_v.at[NCHUNK - NBUF + b]],
                        add=True)

    plsc.subcore_barrier()
    pltpu.sync_copy(agg_sh.at[pl.ds(s * RPT, RPT)],
                    out_hbm.at[c, pl.ds(s * RPT, RPT)])


def _make_agg(F):
    return pl.kernel(
        _agg_body,
        out_type=jax.ShapeDtypeStruct((NC, NPAD, F), jnp.float32),
        mesh=plsc.VectorSubcoreMesh(core_axis_name="c", subcore_axis_name="s"),
        compiler_params=pltpu.CompilerParams(use_tc_tiling_on_sc=False),
        scratch_types=[
            pltpu.VMEM((NCHUNK, CH), jnp.int32),
            pltpu.VMEM((NCHUNK, CH), jnp.int32),
            pltpu.VMEM((NBUF, CH, F), jnp.float32),
            pltpu.VMEM_SHARED((NPAD, F), jnp.float32),
            pltpu.SemaphoreType.DMA((NBUF,)),
        ],
    )


_agg32 = _make_agg(32)
_agg64 = _make_agg(64)


def _tc1_body(x_ref, w1_ref, p_ref, xw1_ref, dinv_ref, xs1_ref):
    xw1 = jnp.dot(x_ref[...], w1_ref[...], preferred_element_type=jnp.float32)
    p = p_ref[...]
    deg = p[0] + p[1] + 1.0
    dinv = lax.rsqrt(deg)[:, None]
    row = lax.broadcasted_iota(jnp.int32, (NPAD, 1), 0)
    dinv = jnp.where(row < N, dinv, 0.0)
    xw1_ref[...] = xw1
    dinv_ref[...] = dinv
    xs1_ref[...] = xw1 * dinv


_tc1 = pl.pallas_call(
    _tc1_body,
    out_shape=(
        jax.ShapeDtypeStruct((NPAD, 32), jnp.float32),
        jax.ShapeDtypeStruct((NPAD, 1), jnp.float32),
        jax.ShapeDtypeStruct((NPAD, 32), jnp.float32),
    ),
)


def _tc2_body(q_ref, xw1_ref, dinv_ref, b1_ref, w2_ref, xw2_ref, xs2_ref):
    q = q_ref[...]
    dinv = dinv_ref[...]
    pre = dinv * (q[0] + q[1]) + dinv * dinv * xw1_ref[...] + b1_ref[...]
    h1 = jnp.where(pre >= 0, pre, 0.1 * pre)
    xw2 = jnp.dot(h1, w2_ref[...], preferred_element_type=jnp.float32)
    xw2_ref[...] = xw2
    xs2_ref[...] = xw2 * dinv


_tc2 = pl.pallas_call(
    _tc2_body,
    out_shape=(
        jax.ShapeDtypeStruct((NPAD, 64), jnp.float32),
        jax.ShapeDtypeStruct((NPAD, 64), jnp.float32),
    ),
)


def _tc3_body(r_ref, xw2_ref, dinv_ref, b2_ref, batch_ref,
              l1w_ref, l1b_ref, l2w_ref, l2b_ref, y_ref, pooled_ref):
    rr = r_ref[...]
    dinv = dinv_ref[...]
    h2 = dinv * (rr[0] + rr[1]) + dinv * dinv * xw2_ref[...] + b2_ref[...]
    b = batch_ref[...]

    def _seg(g, carry):
        v = jnp.where(b == g, h2, -jnp.inf)
        pooled_ref[pl.ds(g, 1), :] = jnp.max(v, axis=0, keepdims=True)
        return carry

    lax.fori_loop(0, NG, _seg, 0)
    pooled = pooled_ref[...]
    t = jnp.dot(pooled, l1w_ref[...], preferred_element_type=jnp.float32) + l1b_ref[...]
    t = jnp.where(t >= 0, t, 0.1 * t)
    y_ref[...] = jnp.dot(t, l2w_ref[...], preferred_element_type=jnp.float32) + l2b_ref[...]


_tc3 = pl.pallas_call(
    _tc3_body,
    out_shape=jax.ShapeDtypeStruct((NG, 32), jnp.float32),
    scratch_shapes=[pltpu.VMEM((NG, 64), jnp.float32)],
)


def kernel(x, edge_index, batch, W1, b1, W2, b2, L1W, L1b, L2W, L2b):
    edge_index = edge_index.astype(jnp.int32)
    src = edge_index[0]
    dst = edge_index[1]
    pad = jnp.full((EPAD - E,), N, dtype=jnp.int32)
    src_r = jnp.concatenate([src, pad]).reshape(NW, NCHUNK, CH)
    dst_r = jnp.concatenate([dst, pad]).reshape(NW, NCHUNK, CH)
    aux = jnp.concatenate([jnp.ones((CH,), jnp.float32),
                           jnp.zeros((RPT,), jnp.float32)])
    x_pad = jnp.pad(x, ((0, NPAD - N), (0, 0)))
    batch_pad = jnp.concatenate(
        [batch.astype(jnp.int32), jnp.full((NPAD - N,), NG, jnp.int32)])[:, None]

    p = _deg_call(dst_r, aux)
    xw1, dinv, xs1 = _tc1(x_pad, W1, p)
    q = _agg32(xs1, src_r, dst_r)
    xw2, xs2 = _tc2(q, xw1, dinv, b1, W2)
    r = _agg64(xs2, src_r, dst_r)
    y = _tc3(r, xw2, dinv, b2, batch_pad, L1W, L1b, L2W, L2b)
    return y


# speedup vs baseline: 17.0144x; 1.3291x over previous
import jax
import jax.numpy as jnp
from jax import lax
from jax.experimental import pallas as pl
from jax.experimental.pallas import tpu as pltpu
from jax.experimental.pallas import tpu_sc as plsc

N = 10000
NPAD = 10240
E = 160000
NG = 64
NC = 2
NS = 16
NW = NC * NS
CH = 128
NCHUNK = 40
EPT = CH * NCHUNK
EPAD = EPT * NW
RPT = NPAD // NS
ZOFF = NPAD - CH


def _deg_body(dst_hbm, aux_hbm, out_hbm, dst_v, ones_v, deg_sh):
    c = lax.axis_index("c")
    s = lax.axis_index("s")
    wid = c * NS + s
    pltpu.sync_copy(dst_hbm.at[wid], dst_v)
    pltpu.sync_copy(aux_hbm.at[pl.ds(0, CH)], ones_v)
    pltpu.sync_copy(aux_hbm.at[pl.ds(CH, RPT)], deg_sh.at[pl.ds(s * RPT, RPT)])
    plsc.subcore_barrier()

    @pl.loop(0, NCHUNK)
    def _chunk(j):
        pltpu.sync_copy(ones_v, deg_sh.at[dst_v.at[j]], add=True)

    plsc.subcore_barrier()
    pltpu.sync_copy(deg_sh.at[pl.ds(s * RPT, RPT)],
                    out_hbm.at[c, pl.ds(s * RPT, RPT)])


_deg_call = pl.kernel(
    _deg_body,
    out_type=jax.ShapeDtypeStruct((NC, NPAD), jnp.float32),
    mesh=plsc.VectorSubcoreMesh(core_axis_name="c", subcore_axis_name="s"),
    scratch_types=[
        pltpu.VMEM((NCHUNK, CH), jnp.int32),
        pltpu.VMEM((CH,), jnp.float32),
        pltpu.VMEM_SHARED((NPAD,), jnp.float32),
    ],
)


NBUF = 4


def _agg_body(xs_hbm, src_hbm, dst_hbm, out_hbm, src_v, dst_v, rows_v, agg_sh,
              xs_sh, gsem):
    c = lax.axis_index("c")
    s = lax.axis_index("s")
    wid = c * NS + s
    pltpu.sync_copy(src_hbm.at[wid], src_v)
    pltpu.sync_copy(dst_hbm.at[wid], dst_v)
    pltpu.sync_copy(xs_hbm.at[pl.ds(s * RPT, RPT)],
                    xs_sh.at[pl.ds(s * RPT, RPT)])
    for k in range(RPT // CH):
        pltpu.sync_copy(xs_hbm.at[pl.ds(ZOFF, CH)],
                        agg_sh.at[pl.ds(s * RPT + k * CH, CH)])
    plsc.subcore_barrier()

    def _start_gather(j, b):
        pltpu.async_copy(xs_sh.at[src_v.at[j]], rows_v.at[b], gsem.at[b])

    def _wait_gather(b):
        pltpu.make_async_copy(xs_sh.at[src_v.at[0]], rows_v.at[b],
                              gsem.at[b]).wait()

    for b in range(NBUF):
        _start_gather(b, b)

    @pl.loop(0, NCHUNK - NBUF, step=NBUF)
    def _round(j0):
        for b in range(NBUF):
            j = j0 + b
            _wait_gather(b)
            pltpu.sync_copy(rows_v.at[b], agg_sh.at[dst_v.at[j]], add=True)
            _start_gather(j + NBUF, b)

    for b in range(NBUF):
        _wait_gather(b)
        pltpu.sync_copy(rows_v.at[b], agg_sh.at[dst_v.at[NCHUNK - NBUF + b]],
                        add=True)

    plsc.subcore_barrier()
    pltpu.sync_copy(agg_sh.at[pl.ds(s * RPT, RPT)],
                    out_hbm.at[c, pl.ds(s * RPT, RPT)])


def _make_agg(F):
    return pl.kernel(
        _agg_body,
        out_type=jax.ShapeDtypeStruct((NC, NPAD, F), jnp.float32),
        mesh=plsc.VectorSubcoreMesh(core_axis_name="c", subcore_axis_name="s"),
        compiler_params=pltpu.CompilerParams(use_tc_tiling_on_sc=False),
        scratch_types=[
            pltpu.VMEM((NCHUNK, CH), jnp.int32),
            pltpu.VMEM((NCHUNK, CH), jnp.int32),
            pltpu.VMEM((NBUF, CH, F), jnp.float32),
            pltpu.VMEM_SHARED((NPAD, F), jnp.float32),
            pltpu.VMEM_SHARED((NPAD, F), jnp.float32),
            pltpu.SemaphoreType.DMA((NBUF,)),
        ],
    )


_agg32 = _make_agg(32)
_agg64 = _make_agg(64)


def _tc1_body(x_ref, w1_ref, p_ref, xw1_ref, dinv_ref, xs1_ref):
    xw1 = jnp.dot(x_ref[...], w1_ref[...], preferred_element_type=jnp.float32)
    p = p_ref[...]
    deg = p[0] + p[1] + 1.0
    dinv = lax.rsqrt(deg)[:, None]
    row = lax.broadcasted_iota(jnp.int32, (NPAD, 1), 0)
    dinv = jnp.where(row < N, dinv, 0.0)
    xw1_ref[...] = xw1
    dinv_ref[...] = dinv
    xs1_ref[...] = xw1 * dinv


_tc1 = pl.pallas_call(
    _tc1_body,
    out_shape=(
        jax.ShapeDtypeStruct((NPAD, 32), jnp.float32),
        jax.ShapeDtypeStruct((NPAD, 1), jnp.float32),
        jax.ShapeDtypeStruct((NPAD, 32), jnp.float32),
    ),
)


def _tc2_body(q_ref, xw1_ref, dinv_ref, b1_ref, w2_ref, xw2_ref, xs2_ref):
    q = q_ref[...]
    dinv = dinv_ref[...]
    pre = dinv * (q[0] + q[1]) + dinv * dinv * xw1_ref[...] + b1_ref[...]
    h1 = jnp.where(pre >= 0, pre, 0.1 * pre)
    xw2 = jnp.dot(h1, w2_ref[...], preferred_element_type=jnp.float32)
    xw2_ref[...] = xw2
    xs2_ref[...] = xw2 * dinv


_tc2 = pl.pallas_call(
    _tc2_body,
    out_shape=(
        jax.ShapeDtypeStruct((NPAD, 64), jnp.float32),
        jax.ShapeDtypeStruct((NPAD, 64), jnp.float32),
    ),
)


def _tc3_body(r_ref, xw2_ref, dinv_ref, b2_ref, batch_ref,
              l1w_ref, l1b_ref, l2w_ref, l2b_ref, y_ref, pooled_ref):
    rr = r_ref[...]
    dinv = dinv_ref[...]
    h2 = dinv * (rr[0] + rr[1]) + dinv * dinv * xw2_ref[...] + b2_ref[...]
    b = batch_ref[...]

    def _seg(g, carry):
        v = jnp.where(b == g, h2, -jnp.inf)
        pooled_ref[pl.ds(g, 1), :] = jnp.max(v, axis=0, keepdims=True)
        return carry

    lax.fori_loop(0, NG, _seg, 0)
    pooled = pooled_ref[...]
    t = jnp.dot(pooled, l1w_ref[...], preferred_element_type=jnp.float32) + l1b_ref[...]
    t = jnp.where(t >= 0, t, 0.1 * t)
    y_ref[...] = jnp.dot(t, l2w_ref[...], preferred_element_type=jnp.float32) + l2b_ref[...]


_tc3 = pl.pallas_call(
    _tc3_body,
    out_shape=jax.ShapeDtypeStruct((NG, 32), jnp.float32),
    scratch_shapes=[pltpu.VMEM((NG, 64), jnp.float32)],
)


def kernel(x, edge_index, batch, W1, b1, W2, b2, L1W, L1b, L2W, L2b):
    edge_index = edge_index.astype(jnp.int32)
    src = edge_index[0]
    dst = edge_index[1]
    pad = jnp.full((EPAD - E,), N, dtype=jnp.int32)
    src_r = jnp.concatenate([src, pad]).reshape(NW, NCHUNK, CH)
    dst_r = jnp.concatenate([dst, pad]).reshape(NW, NCHUNK, CH)
    aux = jnp.concatenate([jnp.ones((CH,), jnp.float32),
                           jnp.zeros((RPT,), jnp.float32)])
    x_pad = jnp.pad(x, ((0, NPAD - N), (0, 0)))
    batch_pad = jnp.concatenate(
        [batch.astype(jnp.int32), jnp.full((NPAD - N,), NG, jnp.int32)])[:, None]

    p = _deg_call(dst_r, aux)
    xw1, dinv, xs1 = _tc1(x_pad, W1, p)
    q = _agg32(xs1, src_r, dst_r)
    xw2, xs2 = _tc2(q, xw1, dinv, b1, W2)
    r = _agg64(xs2, src_r, dst_r)
    y = _tc3(r, xw2, dinv, b2, batch_pad, L1W, L1b, L2W, L2b)
    return y


# speedup vs baseline: 25.6605x; 1.5082x over previous
import jax
import jax.numpy as jnp
from jax import lax
from jax.experimental import pallas as pl
from jax.experimental.pallas import tpu as pltpu
from jax.experimental.pallas import tpu_sc as plsc

N = 10000
NPAD = 10240
E = 160000
NG = 64
NGP = 72
NC = 2
NS = 16
CH = 128
NCHUNK = 80
EPT = CH * NCHUNK
EPAD = EPT * NS
RPT = NPAD // NS
ZOFF = NPAD - CH
NBUF = 4
DEGC = NCHUNK // NC


def _deg_body(dst_hbm, aux_hbm, out_hbm, dst_v, ones_v, deg_sh):
    c = lax.axis_index("c")
    s = lax.axis_index("s")
    pltpu.sync_copy(dst_hbm.at[s, pl.ds(c * DEGC, DEGC)], dst_v)
    pltpu.sync_copy(aux_hbm.at[pl.ds(0, CH)], ones_v)
    pltpu.sync_copy(aux_hbm.at[pl.ds(CH, RPT)], deg_sh.at[pl.ds(s * RPT, RPT)])
    plsc.subcore_barrier()

    @pl.loop(0, DEGC)
    def _chunk(j):
        pltpu.sync_copy(ones_v, deg_sh.at[dst_v.at[j]], add=True)

    plsc.subcore_barrier()
    pltpu.sync_copy(deg_sh.at[pl.ds(s * RPT, RPT)],
                    out_hbm.at[c, pl.ds(s * RPT, RPT)])


_deg_call = pl.kernel(
    _deg_body,
    out_type=jax.ShapeDtypeStruct((NC, NPAD), jnp.float32),
    mesh=plsc.VectorSubcoreMesh(core_axis_name="c", subcore_axis_name="s"),
    scratch_types=[
        pltpu.VMEM((DEGC, CH), jnp.int32),
        pltpu.VMEM((CH,), jnp.float32),
        pltpu.VMEM_SHARED((NPAD,), jnp.float32),
    ],
)


def _agg_pipeline(xs_hbm, src_hbm, dst_hbm, src_v, dst_v, rows_v, agg_sh,
                  xs_sh, gsem, c, s, F2):
    pltpu.sync_copy(src_hbm.at[s], src_v)
    pltpu.sync_copy(dst_hbm.at[s], dst_v)
    pltpu.sync_copy(xs_hbm.at[pl.ds(s * RPT, RPT), pl.ds(c * F2, F2)],
                    xs_sh.at[pl.ds(s * RPT, RPT)])
    for k in range(RPT // CH):
        pltpu.sync_copy(xs_hbm.at[pl.ds(ZOFF, CH), pl.ds(c * F2, F2)],
                        agg_sh.at[pl.ds(s * RPT + k * CH, CH)])
    plsc.subcore_barrier()

    def _start_gather(j, b):
        pltpu.async_copy(xs_sh.at[src_v.at[j]], rows_v.at[b], gsem.at[b])

    def _wait_gather(b):
        pltpu.make_async_copy(xs_sh.at[src_v.at[0]], rows_v.at[b],
                              gsem.at[b]).wait()

    for b in range(NBUF):
        _start_gather(b, b)

    @pl.loop(0, NCHUNK - NBUF, step=NBUF)
    def _round(j0):
        for b in range(NBUF):
            j = j0 + b
            _wait_gather(b)
            pltpu.sync_copy(rows_v.at[b], agg_sh.at[dst_v.at[j]], add=True)
            _start_gather(j + NBUF, b)

    for b in range(NBUF):
        _wait_gather(b)
        pltpu.sync_copy(rows_v.at[b], agg_sh.at[dst_v.at[NCHUNK - NBUF + b]],
                        add=True)

    plsc.subcore_barrier()


def _agg32_body(xs_hbm, src_hbm, dst_hbm, out_hbm, src_v, dst_v, rows_v,
                agg_sh, xs_sh, gsem):
    c = lax.axis_index("c")
    s = lax.axis_index("s")
    _agg_pipeline(xs_hbm, src_hbm, dst_hbm, src_v, dst_v, rows_v, agg_sh,
                  xs_sh, gsem, c, s, 16)
    pltpu.sync_copy(agg_sh.at[pl.ds(s * RPT, RPT)],
                    out_hbm.at[c, pl.ds(s * RPT, RPT)])


_agg32_call = pl.kernel(
    _agg32_body,
    out_type=jax.ShapeDtypeStruct((NC, NPAD, 16), jnp.float32),
    mesh=plsc.VectorSubcoreMesh(core_axis_name="c", subcore_axis_name="s"),
    compiler_params=pltpu.CompilerParams(use_tc_tiling_on_sc=False),
    scratch_types=[
        pltpu.VMEM((NCHUNK, CH), jnp.int32),
        pltpu.VMEM((NCHUNK, CH), jnp.int32),
        pltpu.VMEM((NBUF, CH, 16), jnp.float32),
        pltpu.VMEM_SHARED((NPAD, 16), jnp.float32),
        pltpu.VMEM_SHARED((NPAD, 16), jnp.float32),
        pltpu.SemaphoreType.DMA((NBUF,)),
    ],
)


def _agg64_body(xs_hbm, src_hbm, dst_hbm, xw2_hbm, dinv_hbm, b2_hbm,
                batch_hbm, out_hbm, src_v, dst_v, rows_v, agg_sh, xs_sh,
                gsem, agg_v, xw_v, dinv_v, batch_v, b2_v, acc_v):
    F2 = 32
    c = lax.axis_index("c")
    s = lax.axis_index("s")
    _agg_pipeline(xs_hbm, src_hbm, dst_hbm, src_v, dst_v, rows_v, agg_sh,
                  xs_sh, gsem, c, s, F2)

    pltpu.sync_copy(agg_sh.at[pl.ds(s * RPT, RPT)], agg_v)
    pltpu.sync_copy(xw2_hbm.at[pl.ds(s * RPT, RPT), pl.ds(c * F2, F2)], xw_v)
    pltpu.sync_copy(dinv_hbm.at[pl.ds(s * RPT, RPT)], dinv_v)
    pltpu.sync_copy(batch_hbm.at[pl.ds(s * RPT, RPT)], batch_v)
    pltpu.sync_copy(b2_hbm.at[c], b2_v)

    neg = jnp.full((16,), -jnp.inf, dtype=jnp.float32)

    @pl.loop(0, NGP)
    def _init(g):
        for k in range(F2 // 16):
            acc_v[g, pl.ds(k * 16, 16)] = neg

    b2a = b2_v[pl.ds(0, 16)]
    b2b = b2_v[pl.ds(16, 16)]

    @pl.loop(0, RPT, step=16)
    def _row(r0):
        d16 = dinv_v[pl.ds(r0, 16)]
        g16 = batch_v[pl.ds(r0, 16)]
        for i in range(16):
            r = r0 + i
            d = d16[i]
            dd = d * d
            g = g16[i]
            ha = d * agg_v[r, pl.ds(0, 16)] + dd * xw_v[r, pl.ds(0, 16)] + b2a
            hb = d * agg_v[r, pl.ds(16, 16)] + dd * xw_v[r, pl.ds(16, 16)] + b2b
            acc_v[g, pl.ds(0, 16)] = jnp.maximum(acc_v[g, pl.ds(0, 16)], ha)
            acc_v[g, pl.ds(16, 16)] = jnp.maximum(acc_v[g, pl.ds(16, 16)], hb)

    pltpu.sync_copy(acc_v, out_hbm.at[c, s])


_agg64_call = pl.kernel(
    _agg64_body,
    out_type=jax.ShapeDtypeStruct((NC, NS, NGP, 32), jnp.float32),
    mesh=plsc.VectorSubcoreMesh(core_axis_name="c", subcore_axis_name="s"),
    compiler_params=pltpu.CompilerParams(use_tc_tiling_on_sc=False),
    scratch_types=[
        pltpu.VMEM((NCHUNK, CH), jnp.int32),
        pltpu.VMEM((NCHUNK, CH), jnp.int32),
        pltpu.VMEM((NBUF, CH, 32), jnp.float32),
        pltpu.VMEM_SHARED((NPAD, 32), jnp.float32),
        pltpu.VMEM_SHARED((NPAD, 32), jnp.float32),
        pltpu.SemaphoreType.DMA((NBUF,)),
        pltpu.VMEM((RPT, 32), jnp.float32),
        pltpu.VMEM((RPT, 32), jnp.float32),
        pltpu.VMEM((RPT,), jnp.float32),
        pltpu.VMEM((RPT,), jnp.int32),
        pltpu.VMEM((32,), jnp.float32),
        pltpu.VMEM((NGP, 32), jnp.float32),
    ],
)


def _tc1_body(x_ref, w1_ref, p_ref, xw1_ref, dinv_ref, xs1_ref):
    xw1 = jnp.dot(x_ref[...], w1_ref[...], preferred_element_type=jnp.float32)
    p = p_ref[...]
    deg = p[0] + p[1] + 1.0
    dinv = lax.rsqrt(deg)[:, None]
    row = lax.broadcasted_iota(jnp.int32, (NPAD, 1), 0)
    dinv = jnp.where(row < N, dinv, 0.0)
    xw1_ref[...] = xw1
    dinv_ref[...] = dinv
    xs1_ref[...] = xw1 * dinv


_tc1 = pl.pallas_call(
    _tc1_body,
    out_shape=(
        jax.ShapeDtypeStruct((NPAD, 32), jnp.float32),
        jax.ShapeDtypeStruct((NPAD, 1), jnp.float32),
        jax.ShapeDtypeStruct((NPAD, 32), jnp.float32),
    ),
)


def _tc2_body(q_ref, xw1_ref, dinv_ref, b1_ref, w2_ref, xw2_ref, xs2_ref):
    q = q_ref[...]
    agg1 = jnp.concatenate([q[0], q[1]], axis=-1)
    dinv = dinv_ref[...]
    pre = dinv * agg1 + dinv * dinv * xw1_ref[...] + b1_ref[...]
    h1 = jnp.where(pre >= 0, pre, 0.1 * pre)
    xw2 = jnp.dot(h1, w2_ref[...], preferred_element_type=jnp.float32)
    xw2_ref[...] = xw2
    xs2_ref[...] = xw2 * dinv


_tc2 = pl.pallas_call(
    _tc2_body,
    out_shape=(
        jax.ShapeDtypeStruct((NPAD, 64), jnp.float32),
        jax.ShapeDtypeStruct((NPAD, 64), jnp.float32),
    ),
)


def _tc3_body(m_ref, l1w_ref, l1b_ref, l2w_ref, l2b_ref, y_ref):
    m = m_ref[...]
    pooled = jnp.concatenate([jnp.max(m[0, :, :NG, :], axis=0),
                              jnp.max(m[1, :, :NG, :], axis=0)], axis=-1)
    t = jnp.dot(pooled, l1w_ref[...], preferred_element_type=jnp.float32) + l1b_ref[...]
    t = jnp.where(t >= 0, t, 0.1 * t)
    y_ref[...] = jnp.dot(t, l2w_ref[...], preferred_element_type=jnp.float32) + l2b_ref[...]


_tc3 = pl.pallas_call(
    _tc3_body,
    out_shape=jax.ShapeDtypeStruct((NG, 32), jnp.float32),
)


def kernel(x, edge_index, batch, W1, b1, W2, b2, L1W, L1b, L2W, L2b):
    edge_index = edge_index.astype(jnp.int32)
    src = edge_index[0]
    dst = edge_index[1]
    pad = jnp.full((EPAD - E,), N, dtype=jnp.int32)
    src_r = jnp.concatenate([src, pad]).reshape(NS, NCHUNK, CH)
    dst_r = jnp.concatenate([dst, pad]).reshape(NS, NCHUNK, CH)
    aux = jnp.concatenate([jnp.ones((CH,), jnp.float32),
                           jnp.zeros((RPT,), jnp.float32)])
    x_pad = jnp.pad(x, ((0, NPAD - N), (0, 0)))
    batch_pad = jnp.concatenate(
        [batch.astype(jnp.int32), jnp.full((NPAD - N,), NG, jnp.int32)])
    b2_r = b2.reshape(NC, 32)

    p = _deg_call(dst_r, aux)
    xw1, dinv, xs1 = _tc1(x_pad, W1, p)
    q = _agg32_call(xs1, src_r, dst_r)
    xw2, xs2 = _tc2(q, xw1, dinv, b1, W2)
    m = _agg64_call(xs2, src_r, dst_r, xw2, dinv[:, 0], b2_r, batch_pad)
    y = _tc3(m, L1W, L1b, L2W, L2b)
    return y
